# baseline (speedup 1.0000x reference)
import jax
import jax.numpy as jnp
from jax.experimental import pallas as pl

B = 4096
F = 16
D_EMB = 64
H = 128
HEADS = 4
C = 32
N = B * F
E = 2 * N
U = D_EMB + 1

ROWS_BLK = 2048


def _attn1_body(u_ref, w_ref, o_ref):
    o_ref[...] = jnp.dot(u_ref[...], w_ref[...],
                         preferred_element_type=jnp.float32)


def _attn1(u, AWsd):
    return pl.pallas_call(
        _attn1_body,
        grid=(N // ROWS_BLK,),
        in_specs=[
            pl.BlockSpec((ROWS_BLK, U), lambda i: (i, 0)),
            pl.BlockSpec((U, 2 * HEADS), lambda i: (0, 0)),
        ],
        out_specs=pl.BlockSpec((ROWS_BLK, 2 * HEADS), lambda i: (i, 0)),
        out_shape=jax.ShapeDtypeStruct((N, 2 * HEADS), jnp.float32),
    )(u, AWsd)


def _l1post_body(numu_ref, den_ref, u_ref, ws_ref, bd_ref, awc_ref, b1_ref,
                 w2_ref, o_ref):
    ymm = jnp.dot(numu_ref[...], bd_ref[...],
                  preferred_element_type=jnp.float32)
    z = jnp.dot(u_ref[...], awc_ref[...],
                preferred_element_type=jnp.float32)
    den = den_ref[...]
    ws = ws_ref[...]
    outs = []
    for h in range(HEADS):
        sl = slice(h * H, (h + 1) * H)
        outs.append((ymm[:, sl] + ws[:, h:h + 1] * z[:, sl])
                    / (den[:, h:h + 1] + ws[:, h:h + 1]))
    y = jnp.concatenate(outs, axis=1) + b1_ref[...]
    x2 = jnp.where(y > 0, y, jnp.exp(jnp.minimum(y, 0.0)) - 1.0)
    o_ref[...] = jnp.dot(x2, w2_ref[...], preferred_element_type=jnp.float32)


def _l1post(numu, den, u, wself1, BD, AWcat, b1, W2):
    R = 1024
    return pl.pallas_call(
        _l1post_body,
        grid=(N // R,),
        in_specs=[
            pl.BlockSpec((R, HEADS * U), lambda i: (i, 0)),
            pl.BlockSpec((R, HEADS), lambda i: (i, 0)),
            pl.BlockSpec((R, U), lambda i: (i, 0)),
            pl.BlockSpec((R, HEADS), lambda i: (i, 0)),
            pl.BlockSpec((HEADS * U, HEADS * H), lambda i: (0, 0)),
            pl.BlockSpec((U, HEADS * H), lambda i: (0, 0)),
            pl.BlockSpec((1, HEADS * H), lambda i: (0, 0)),
            pl.BlockSpec((HEADS * H, H), lambda i: (0, 0)),
        ],
        out_specs=pl.BlockSpec((R, H), lambda i: (i, 0)),
        out_shape=jax.ShapeDtypeStruct((N, H), jnp.float32),
    )(numu, den, u, wself1, BD, AWcat, b1[None, :], W2)


def _cls_body(s_ref, c_ref, w1_ref, b1_ref, w2_ref, b2_ref, o_ref):
    ge = s_ref[...] / jnp.maximum(c_ref[...], 1.0)
    hc = jnp.dot(ge, w1_ref[...], preferred_element_type=jnp.float32) \
        + b1_ref[...]
    hc = jnp.where(hc > 0, hc, 0.01 * hc)
    o_ref[...] = jnp.dot(hc, w2_ref[...],
                         preferred_element_type=jnp.float32) + b2_ref[...]


def _classifier(sums, counts, Wc1, bc1, Wc2, bc2):
    R = 1024
    return pl.pallas_call(
        _cls_body,
        grid=(B // R,),
        in_specs=[
            pl.BlockSpec((R, H), lambda i: (i, 0)),
            pl.BlockSpec((R, 1), lambda i: (i, 0)),
            pl.BlockSpec((H, H), lambda i: (0, 0)),
            pl.BlockSpec((1, H), lambda i: (0, 0)),
            pl.BlockSpec((H, C), lambda i: (0, 0)),
            pl.BlockSpec((1, C), lambda i: (0, 0)),
        ],
        out_specs=pl.BlockSpec((R, C), lambda i: (i, 0)),
        out_shape=jax.ShapeDtypeStruct((B, C), jnp.float32),
    )(sums, counts, Wc1, bc1[None, :], Wc2, bc2[None, :])


def kernel(embedded, edge_index, batch_idx, W_align, b_align, mask_logits,
           W1, a1_src, a1_dst, b1, W2, a2_src, a2_dst, b2,
           Wc1, bc1, Wc2, bc2):
    gate = jax.nn.sigmoid(mask_logits)
    emb2d = embedded.reshape(N, D_EMB)
    gnode = jnp.tile(gate, B)
    u = jnp.concatenate([emb2d * gnode[:, None], gnode[:, None]], axis=1)

    src = edge_index[0]
    dst = edge_index[1]

    A = jnp.concatenate([W_align, b_align[None, :]], axis=0)
    W1r = W1.reshape(H, HEADS, H)
    Ws1 = jnp.einsum("dhk,hk->dh", W1r, a1_src)
    Wd1 = jnp.einsum("dhk,hk->dh", W1r, a1_dst)
    AWsd = A @ jnp.concatenate([Ws1, Wd1], axis=1)
    att = _attn1(u, AWsd)
    asrc1, adst1 = att[:, :HEADS], att[:, HEADS:]

    AWcat = A @ W1
    BD = jnp.zeros((HEADS * U, HEADS * H), jnp.float32)
    for h in range(HEADS):
        BD = BD.at[h * U:(h + 1) * U, h * H:(h + 1) * H].set(
            AWcat[:, h * H:(h + 1) * H])

    t1 = asrc1 + adst1
    wself1 = jnp.exp(jnp.where(t1 >= 0, t1, 0.2 * t1))
    gsrc = jnp.concatenate([u, asrc1], axis=1)[src]
    te = gsrc[:, U:] + adst1[dst]
    w1e = jnp.exp(jnp.where(te >= 0, te, 0.2 * te))
    msg1 = jnp.concatenate(
        [(w1e[:, :, None] * gsrc[:, None, :U]).reshape(E, HEADS * U), w1e],
        axis=1)
    agg1 = jax.ops.segment_sum(msg1, dst, num_segments=N)
    xl2 = _l1post(agg1[:, :HEADS * U], agg1[:, HEADS * U:], u, wself1,
                  BD, AWcat, b1, W2)

    asrc2 = xl2 @ a2_src[0]
    adst2 = xl2 @ a2_dst[0]
    t2 = asrc2 + adst2
    wself2 = jnp.exp(jnp.where(t2 >= 0, t2, 0.2 * t2))
    gsrc2 = jnp.concatenate([xl2, asrc2[:, None]], axis=1)[src]
    te2 = gsrc2[:, H] + adst2[dst]
    w2e = jnp.exp(jnp.where(te2 >= 0, te2, 0.2 * te2))
    msg2 = jnp.concatenate([w2e[:, None] * gsrc2[:, :H], w2e[:, None]],
                           axis=1)
    agg2 = jax.ops.segment_sum(msg2, dst, num_segments=N)
    num2, den2 = agg2[:, :H], agg2[:, H]
    x3 = (num2 + wself2[:, None] * xl2) / (den2 + wself2)[:, None] + b2

    pooled = jax.ops.segment_sum(
        jnp.concatenate([x3, jnp.ones((N, 1), jnp.float32)], axis=1),
        batch_idx, num_segments=B, indices_are_sorted=True)

    logits = _classifier(pooled[:, :H], pooled[:, H:], Wc1, bc1, Wc2, bc2)
    return (logits, gate)


# speedup vs baseline: 1.0120x; 1.0120x over previous
import jax
import jax.numpy as jnp
from jax.experimental import pallas as pl

B = 4096
F = 16
D_EMB = 64
H = 128
HEADS = 4
C = 32
N = B * F
E = 2 * N
U = D_EMB + 1

ROWS_BLK = 2048


def _attn1_body(u_ref, w_ref, o_ref):
    o_ref[...] = jnp.dot(u_ref[...], w_ref[...],
                         preferred_element_type=jnp.float32)


def _attn1(u, AWsd):
    return pl.pallas_call(
        _attn1_body,
        grid=(N // ROWS_BLK,),
        in_specs=[
            pl.BlockSpec((ROWS_BLK, U), lambda i: (i, 0)),
            pl.BlockSpec((U, 2 * HEADS), lambda i: (0, 0)),
        ],
        out_specs=pl.BlockSpec((ROWS_BLK, 2 * HEADS), lambda i: (i, 0)),
        out_shape=jax.ShapeDtypeStruct((N, 2 * HEADS), jnp.float32),
    )(u, AWsd)


def _l1post_body(numu_ref, den_ref, u_ref, ws_ref, bd_ref, awc_ref, b1_ref,
                 w2_ref, o_ref):
    ymm = jnp.dot(numu_ref[...], bd_ref[...],
                  preferred_element_type=jnp.float32)
    z = jnp.dot(u_ref[...], awc_ref[...],
                preferred_element_type=jnp.float32)
    den = den_ref[...]
    ws = ws_ref[...]
    outs = []
    for h in range(HEADS):
        sl = slice(h * H, (h + 1) * H)
        outs.append((ymm[:, sl] + ws[:, h:h + 1] * z[:, sl])
                    / (den[:, h:h + 1] + ws[:, h:h + 1]))
    y = jnp.concatenate(outs, axis=1) + b1_ref[...]
    x2 = jnp.where(y > 0, y, jnp.exp(jnp.minimum(y, 0.0)) - 1.0)
    o_ref[...] = jnp.dot(x2, w2_ref[...], preferred_element_type=jnp.float32)


def _l1post(numu, den, u, wself1, BD, AWcat, b1, W2):
    R = 1024
    return pl.pallas_call(
        _l1post_body,
        grid=(N // R,),
        in_specs=[
            pl.BlockSpec((R, HEADS * U), lambda i: (i, 0)),
            pl.BlockSpec((R, HEADS), lambda i: (i, 0)),
            pl.BlockSpec((R, U), lambda i: (i, 0)),
            pl.BlockSpec((R, HEADS), lambda i: (i, 0)),
            pl.BlockSpec((HEADS * U, HEADS * H), lambda i: (0, 0)),
            pl.BlockSpec((U, HEADS * H), lambda i: (0, 0)),
            pl.BlockSpec((1, HEADS * H), lambda i: (0, 0)),
            pl.BlockSpec((HEADS * H, H), lambda i: (0, 0)),
        ],
        out_specs=pl.BlockSpec((R, H), lambda i: (i, 0)),
        out_shape=jax.ShapeDtypeStruct((N, H), jnp.float32),
    )(numu, den, u, wself1, BD, AWcat, b1[None, :], W2)


def _cls_body(s_ref, c_ref, w1_ref, b1_ref, w2_ref, b2_ref, o_ref):
    ge = s_ref[...] / jnp.maximum(c_ref[...], 1.0)
    hc = jnp.dot(ge, w1_ref[...], preferred_element_type=jnp.float32) \
        + b1_ref[...]
    hc = jnp.where(hc > 0, hc, 0.01 * hc)
    o_ref[...] = jnp.dot(hc, w2_ref[...],
                         preferred_element_type=jnp.float32) + b2_ref[...]


def _classifier(sums, counts, Wc1, bc1, Wc2, bc2):
    R = 1024
    return pl.pallas_call(
        _cls_body,
        grid=(B // R,),
        in_specs=[
            pl.BlockSpec((R, H), lambda i: (i, 0)),
            pl.BlockSpec((R, 1), lambda i: (i, 0)),
            pl.BlockSpec((H, H), lambda i: (0, 0)),
            pl.BlockSpec((1, H), lambda i: (0, 0)),
            pl.BlockSpec((H, C), lambda i: (0, 0)),
            pl.BlockSpec((1, C), lambda i: (0, 0)),
        ],
        out_specs=pl.BlockSpec((R, C), lambda i: (i, 0)),
        out_shape=jax.ShapeDtypeStruct((B, C), jnp.float32),
    )(sums, counts, Wc1, bc1[None, :], Wc2, bc2[None, :])


def kernel(embedded, edge_index, batch_idx, W_align, b_align, mask_logits,
           W1, a1_src, a1_dst, b1, W2, a2_src, a2_dst, b2,
           Wc1, bc1, Wc2, bc2):
    gate = jax.nn.sigmoid(mask_logits)
    emb2d = embedded.reshape(N, D_EMB)
    gnode = jnp.tile(gate, B)
    u = jnp.concatenate([emb2d * gnode[:, None], gnode[:, None]], axis=1)

    src = edge_index[0]
    dst = edge_index[1]

    A = jnp.concatenate([W_align, b_align[None, :]], axis=0)
    W1r = W1.reshape(H, HEADS, H)
    Ws1 = jnp.einsum("dhk,hk->dh", W1r, a1_src)
    Wd1 = jnp.einsum("dhk,hk->dh", W1r, a1_dst)
    AWsd = A @ jnp.concatenate([Ws1, Wd1], axis=1)
    att = _attn1(u, AWsd)
    asrc1, adst1 = att[:, :HEADS], att[:, HEADS:]

    AWcat = A @ W1
    BD = jnp.zeros((HEADS * U, HEADS * H), jnp.float32)
    for h in range(HEADS):
        BD = BD.at[h * U:(h + 1) * U, h * H:(h + 1) * H].set(
            AWcat[:, h * H:(h + 1) * H])

    t1 = asrc1 + adst1
    wself1 = jnp.exp(jnp.where(t1 >= 0, t1, 0.2 * t1))
    us = jax.lax.optimization_barrier(u[src])
    te = asrc1[src] + adst1[dst]
    w1e = jnp.exp(jnp.where(te >= 0, te, 0.2 * te))
    msg1 = jnp.concatenate(
        [(w1e[:, :, None] * us[:, None, :]).reshape(E, HEADS * U), w1e],
        axis=1)
    agg1 = jax.ops.segment_sum(msg1, dst, num_segments=N)
    xl2 = _l1post(agg1[:, :HEADS * U], agg1[:, HEADS * U:], u, wself1,
                  BD, AWcat, b1, W2)

    asrc2 = xl2 @ a2_src[0]
    adst2 = xl2 @ a2_dst[0]
    t2 = asrc2 + adst2
    wself2 = jnp.exp(jnp.where(t2 >= 0, t2, 0.2 * t2))
    xs = jax.lax.optimization_barrier(xl2[src])
    te2 = asrc2[src] + adst2[dst]
    w2e = jnp.exp(jnp.where(te2 >= 0, te2, 0.2 * te2))
    msg2 = jnp.concatenate([w2e[:, None] * xs, w2e[:, None]], axis=1)
    agg2 = jax.ops.segment_sum(msg2, dst, num_segments=N)
    num2, den2 = agg2[:, :H], agg2[:, H]
    x3 = (num2 + wself2[:, None] * xl2) / (den2 + wself2)[:, None] + b2

    pooled = jax.ops.segment_sum(
        jnp.concatenate([x3, jnp.ones((N, 1), jnp.float32)], axis=1),
        batch_idx, num_segments=B, indices_are_sorted=True)

    logits = _classifier(pooled[:, :H], pooled[:, H:], Wc1, bc1, Wc2, bc2)
    return (logits, gate)
